# Initial kernel scaffold; baseline (speedup 1.0000x reference)
import jax
import jax.numpy as jnp
from jax.experimental import pallas as pl


def kernel(x_hat, rhs, A_ind, A_val, subspace_vectors, mass):
    raise NotImplementedError("write your pallas kernel here")


# speedup vs baseline: 20.5328x; 20.5328x over previous
import jax
import jax.numpy as jnp
from jax import lax
from jax.experimental import pallas as pl
from jax.experimental.pallas import tpu as pltpu
from jax.experimental.pallas import tpu_sc as plsc

_B, _N, _NSOL = 4, 25000, 8
_BN = _B * _N
_NNZ = 3200000
_NW = 32
_PER_W = _NNZ // _NW
_K = 2000
_NCH = _PER_W // _K
_G = _K // 16


def _sc_body(xT_hbm, row_hbm, col_hbm, val_hbm, out_hbm,
             xs_v, row_v, col_v, val_v, out_v):
    cid = lax.axis_index("core")
    sid = lax.axis_index("subcore")
    wid = sid * 2 + cid
    base = wid * _PER_W
    for s in range(_NSOL):
        pltpu.sync_copy(xT_hbm.at[s], xs_v)

        def chunk_body(ch, accs):
            off = base + ch * _K
            pltpu.sync_copy(row_hbm.at[pl.ds(off, _K)], row_v)
            pltpu.sync_copy(col_hbm.at[pl.ds(off, _K)], col_v)
            pltpu.sync_copy(val_hbm.at[pl.ds(off, _K)], val_v)

            def group_body(g, accs):
                a0, a1, a2, a3 = accs
                e = g * 16
                rows = row_v[pl.ds(e, 16)]
                cols = col_v[pl.ds(e, 16)]
                vals = val_v[pl.ds(e, 16)]
                xr = plsc.load_gather(xs_v, [rows])
                xc = plsc.load_gather(xs_v, [cols])
                p = vals * xr * xc
                m1 = rows >= _N
                m2 = rows >= 2 * _N
                m3 = rows >= 3 * _N
                z = jnp.zeros_like(p)
                a0 = a0 + jnp.where(m1, z, p)
                a1 = a1 + jnp.where(m1 & (~m2), p, z)
                a2 = a2 + jnp.where(m2 & (~m3), p, z)
                a3 = a3 + jnp.where(m3, p, z)
                return (a0, a1, a2, a3)

            return lax.fori_loop(0, _G, group_body, accs)

        z16 = jnp.zeros((16,), jnp.float32)
        a0, a1, a2, a3 = lax.fori_loop(0, _NCH, chunk_body, (z16, z16, z16, z16))
        out_v[s, 0, :] = a0
        out_v[s, 1, :] = a1
        out_v[s, 2, :] = a2
        out_v[s, 3, :] = a3
    pltpu.sync_copy(out_v, out_hbm.at[wid])


_sc_energy = pl.kernel(
    _sc_body,
    out_type=jax.ShapeDtypeStruct((_NW, _NSOL, _B, 16), jnp.float32),
    mesh=plsc.VectorSubcoreMesh(core_axis_name="core", subcore_axis_name="subcore"),
    scratch_types=[
        pltpu.VMEM((_BN,), jnp.float32),
        pltpu.VMEM((_K,), jnp.int32),
        pltpu.VMEM((_K,), jnp.int32),
        pltpu.VMEM((_K,), jnp.float32),
        pltpu.VMEM((_NSOL, _B, 16), jnp.float32),
    ],
    compiler_params=pltpu.CompilerParams(needs_layout_passes=False),
)


def _tc_body(xT3_ref, rhsT3_ref, mass_ref, part_ref, out_ref):
    x = xT3_ref[...]
    r = rhsT3_ref[...]
    m = mass_ref[...]
    loadT = jnp.sum(m[None] * r * x, axis=2)
    vol = jnp.sum(m, axis=1)
    ae = jnp.sum(part_ref[...], axis=(0, 3))
    sigma = loadT / jnp.maximum(ae, 0.0001)
    kkt_e = (0.5 * ae * sigma - loadT) * sigma / vol[None, :]
    comp_b = sigma * loadT / vol[None, :]
    loss = 0.5 * (-jnp.mean(comp_b)) + 0.5 * jnp.mean(kkt_e)
    out_ref[...] = jnp.broadcast_to(loss, (1, 1))


_tc_loss = pl.pallas_call(
    _tc_body,
    out_shape=jax.ShapeDtypeStruct((1, 1), jnp.float32),
)


def kernel(x_hat, rhs, A_ind, A_val, subspace_vectors, mass):
    x32 = x_hat.astype(jnp.float32)
    xT = x32.reshape(_BN, _NSOL).T
    A32 = A_ind.astype(jnp.int32)
    part = _sc_energy(xT, A32[0], A32[1], A_val.astype(jnp.float32))
    xT3 = xT.reshape(_NSOL, _B, _N)
    rhsT3 = rhs.astype(jnp.float32).reshape(_BN, _NSOL).T.reshape(_NSOL, _B, _N)
    mass2 = mass.astype(jnp.float32).reshape(_B, _N)
    out = _tc_loss(xT3, rhsT3, mass2, part)
    return out[0, 0]


# speedup vs baseline: 58.5601x; 2.8520x over previous
import jax
import jax.numpy as jnp
from jax import lax
from jax.experimental import pallas as pl
from jax.experimental.pallas import tpu as pltpu
from jax.experimental.pallas import tpu_sc as plsc

_B, _N, _NSOL = 4, 25000, 8
_BN = _B * _N
_NNZ = 3200000
_NW = 32
_PER_W = _NNZ // _NW
_K = 2000
_NCH = _PER_W // _K
_G = _K // 16


def _sc_body(xT_hbm, rowi_hbm, coli_hbm, val_hbm, out_hbm,
             xs_v, rowa_v, cola_v, vala_v, rowb_v, colb_v, valb_v,
             acc_v, sema, semb):
    cid = lax.axis_index("core")
    sid = lax.axis_index("subcore")
    wid = sid * 2 + cid
    base = wid * _PER_W

    z16 = jnp.zeros((16,), jnp.float32)
    for j in range(_NSOL * _B):
        acc_v[pl.ds(j * 16, 16)] = z16

    iota = lax.broadcasted_iota(jnp.int32, (16,), 0)

    def start_chunk(ch, rowv, colv, valv, sem):
        off = base + ch * _K
        pltpu.async_copy(rowi_hbm.at[pl.ds(off, _K)], rowv, sem)
        pltpu.async_copy(coli_hbm.at[pl.ds(off, _K)], colv, sem)
        pltpu.async_copy(val_hbm.at[pl.ds(off, _K)], valv, sem)

    def wait_chunk(ch, rowv, colv, valv, sem):
        off = base + ch * _K
        pltpu.make_async_copy(rowi_hbm.at[pl.ds(off, _K)], rowv, sem).wait()
        pltpu.make_async_copy(coli_hbm.at[pl.ds(off, _K)], colv, sem).wait()
        pltpu.make_async_copy(val_hbm.at[pl.ds(off, _K)], valv, sem).wait()

    def do_chunk(rowv, colv, valv, ibase):
        @plsc.parallel_loop(0, _G, 1, unroll=4)
        def _(g):
            e = g * 16
            rows = rowv[pl.ds(e, 16)]
            cols = colv[pl.ds(e, 16)]
            vals = valv[pl.ds(e, 16)]
            xr = plsc.load_gather(xs_v, [rows])
            xc = plsc.load_gather(xs_v, [cols])
            p = vals * xr * xc
            w1 = jnp.where(rows >= _N, 16, 0)
            w2 = jnp.where(rows >= 2 * _N, 16, 0)
            w3 = jnp.where(rows >= 3 * _N, 16, 0)
            idx = ibase + w1 + w2 + w3
            plsc.addupdate_scatter(acc_v, [idx], p)

    for s in range(_NSOL):
        pltpu.sync_copy(xT_hbm.at[s], xs_v)
        ibase = iota + s * 64
        start_chunk(0, rowa_v, cola_v, vala_v, sema)
        start_chunk(1, rowb_v, colb_v, valb_v, semb)

        def pair_body(i, carry):
            del carry
            ch0 = 2 * i
            wait_chunk(ch0, rowa_v, cola_v, vala_v, sema)
            do_chunk(rowa_v, cola_v, vala_v, ibase)

            @pl.when(ch0 + 2 < _NCH)
            def _():
                start_chunk(ch0 + 2, rowa_v, cola_v, vala_v, sema)

            wait_chunk(ch0 + 1, rowb_v, colb_v, valb_v, semb)
            do_chunk(rowb_v, colb_v, valb_v, ibase)

            @pl.when(ch0 + 3 < _NCH)
            def _():
                start_chunk(ch0 + 3, rowb_v, colb_v, valb_v, semb)

            return 0

        lax.fori_loop(0, _NCH // 2, pair_body, 0)
    pltpu.sync_copy(acc_v, out_hbm.at[wid])


_sc_energy = pl.kernel(
    _sc_body,
    out_type=jax.ShapeDtypeStruct((_NW, _NSOL * _B * 16), jnp.float32),
    mesh=plsc.VectorSubcoreMesh(core_axis_name="core", subcore_axis_name="subcore"),
    scratch_types=[
        pltpu.VMEM((_BN,), jnp.float32),
        pltpu.VMEM((_K,), jnp.int32),
        pltpu.VMEM((_K,), jnp.int32),
        pltpu.VMEM((_K,), jnp.float32),
        pltpu.VMEM((_K,), jnp.int32),
        pltpu.VMEM((_K,), jnp.int32),
        pltpu.VMEM((_K,), jnp.float32),
        pltpu.VMEM((_NSOL * _B * 16,), jnp.float32),
        pltpu.SemaphoreType.DMA,
        pltpu.SemaphoreType.DMA,
    ],
    compiler_params=pltpu.CompilerParams(needs_layout_passes=False),
)


def _tc_body(xT3_ref, rhsT3_ref, mass_ref, part_ref, out_ref):
    x = xT3_ref[...]
    r = rhsT3_ref[...]
    m = mass_ref[...]
    loadT = jnp.sum(m[None] * r * x, axis=2)
    vol = jnp.sum(m, axis=1)
    ae = jnp.sum(part_ref[...], axis=(0, 3))
    sigma = loadT / jnp.maximum(ae, 0.0001)
    kkt_e = (0.5 * ae * sigma - loadT) * sigma / vol[None, :]
    comp_b = sigma * loadT / vol[None, :]
    loss = 0.5 * (-jnp.mean(comp_b)) + 0.5 * jnp.mean(kkt_e)
    out_ref[...] = jnp.broadcast_to(loss, (1, 1))


_tc_loss = pl.pallas_call(
    _tc_body,
    out_shape=jax.ShapeDtypeStruct((1, 1), jnp.float32),
)


def kernel(x_hat, rhs, A_ind, A_val, subspace_vectors, mass):
    x32 = x_hat.astype(jnp.float32)
    xT = x32.reshape(_BN, _NSOL).T
    A32 = A_ind.astype(jnp.int32)
    part = _sc_energy(xT, A32[0], A32[1], A_val.astype(jnp.float32))
    xT3 = xT.reshape(_NSOL, _B, _N)
    rhsT3 = rhs.astype(jnp.float32).reshape(_BN, _NSOL).T.reshape(_NSOL, _B, _N)
    mass2 = mass.astype(jnp.float32).reshape(_B, _N)
    out = _tc_loss(xT3, rhsT3, mass2, part.reshape(_NW, _NSOL, _B, 16))
    return out[0, 0]
